# Initial kernel scaffold; baseline (speedup 1.0000x reference)
import jax
import jax.numpy as jnp
from jax.experimental import pallas as pl


def kernel(x, edge_index, edge_weight, w1_z, b1_z, w1_r, b1_r, w1_h, b1_h, w2_z, b2_z, w2_r, b2_r, w2_h, b2_h, lin_w, lin_b):
    raise NotImplementedError("write your pallas kernel here")


# speedup vs baseline: 2.6500x; 2.6500x over previous
import functools

import jax
import jax.numpy as jnp
from jax import lax
from jax.experimental import pallas as pl
from jax.experimental.pallas import tpu as pltpu
from jax.experimental.pallas import tpu_sc as plsc

N = 10000
E = 320000
HID = 64
F_IN = 128
T_STEPS = 12

CHUNK = 128
N_TILES = 16
E_PAD = 79 * 32 * CHUNK
NSLICE = 624
TAIL_BASE = NSLICE * N_TILES
TAIL_ROWS = N - TAIL_BASE
ZROWS = 104


_GATHER_DNUMS = lax.GatherDimensionNumbers(
    offset_dims=(), collapsed_slice_dims=(0,), start_index_map=(0,))


def _bcast_lane(vec16, lane):
    sel = jnp.full((16, 1), lane, dtype=jnp.int32)
    return lax.gather(vec16, sel, _GATHER_DNUMS, slice_sizes=(1,),
                      mode=lax.GatherScatterMode.PROMISE_IN_BOUNDS)


@functools.lru_cache(maxsize=None)
def _make_edge_pass(gw, edge_split):
    w = gw // 2
    n_chunks = E_PAD // CHUNK // (32 if edge_split else 16)
    per_worker = n_chunks * CHUNK
    tab_rows = N if edge_split else 2 * N
    mesh = plsc.VectorSubcoreMesh(core_axis_name="c", subcore_axis_name="s")

    @functools.partial(
        pl.kernel,
        out_type=jax.ShapeDtypeStruct((2 * N, w), jnp.float32),
        mesh=mesh,
        scratch_types=[
            pltpu.VMEM((CHUNK,), jnp.int32),
            pltpu.VMEM((CHUNK,), jnp.int32),
            pltpu.VMEM((CHUNK,), jnp.float32),
            pltpu.VMEM((CHUNK,), jnp.float32),
            pltpu.VMEM((CHUNK, gw), jnp.float32),
            pltpu.VMEM((CHUNK, w), jnp.float32),
            pltpu.VMEM((ZROWS, w), jnp.float32),
            pltpu.VMEM_SHARED((N, w), jnp.float32),
            pltpu.SemaphoreType.DMA,
        ],
    )
    def edge_pass(tab_hbm, row_hbm, col_hbm, no_hbm, ni_hbm, out_hbm,
                  idx_v, col_v, no_v, ni_v, rows_v, msg_v, zbuf, acc_sh,
                  sem):
        c = lax.axis_index("c")
        s = lax.axis_index("s")

        zv = jnp.zeros((16,), jnp.float32)
        for i in range(ZROWS):
            for j in range(w // 16):
                zbuf[i, pl.ds(j * 16, 16)] = zv
        for i in range(NSLICE // ZROWS):
            pltpu.sync_copy(
                zbuf, acc_sh.at[pl.ds(s * NSLICE + i * ZROWS, ZROWS)])

        @pl.when(s == N_TILES - 1)
        def _zero_tail():
            pltpu.sync_copy(zbuf.at[pl.ds(0, TAIL_ROWS)],
                            acc_sh.at[pl.ds(TAIL_BASE, TAIL_ROWS)])

        plsc.subcore_barrier()

        def chunk(k, carry):
            if edge_split:
                base = (c * N_TILES + s) * per_worker + k * CHUNK
                pltpu.sync_copy(row_hbm.at[pl.ds(base, CHUNK)], idx_v)
            else:
                base = s * per_worker + k * CHUNK
                pltpu.sync_copy(row_hbm.at[pl.ds(c * E_PAD + base, CHUNK)],
                                idx_v)
            pltpu.sync_copy(col_hbm.at[pl.ds(base, CHUNK)], col_v)
            pltpu.sync_copy(no_hbm.at[pl.ds(base, CHUNK)], no_v)
            pltpu.sync_copy(ni_hbm.at[pl.ds(base, CHUNK)], ni_v)
            pltpu.async_copy(tab_hbm.at[idx_v], rows_v, sem).wait()
            for g in range(CHUNK // 16):
                no_vec = no_v[pl.ds(g * 16, 16)]
                ni_vec = ni_v[pl.ds(g * 16, 16)]
                for e in range(16):
                    eidx = g * 16 + e
                    nb = _bcast_lane(no_vec, e)
                    ib = _bcast_lane(ni_vec, e)
                    for j in range(w // 16):
                        yo = rows_v[eidx, pl.ds(j * 16, 16)]
                        yi = rows_v[eidx, pl.ds(w + j * 16, 16)]
                        msg_v[eidx, pl.ds(j * 16, 16)] = nb * yo + ib * yi
            pltpu.sync_copy(msg_v, acc_sh.at[col_v], add=True)
            return carry

        lax.fori_loop(0, n_chunks, chunk, 0)
        plsc.subcore_barrier()
        pltpu.sync_copy(
            acc_sh.at[pl.ds(s * NSLICE, NSLICE)],
            out_hbm.at[pl.ds(c * N + s * NSLICE, NSLICE)])

        @pl.when(s == N_TILES - 1)
        def _write_tail():
            pltpu.sync_copy(
                acc_sh.at[pl.ds(TAIL_BASE, TAIL_ROWS)],
                out_hbm.at[pl.ds(c * N + TAIL_BASE, TAIL_ROWS)])

    return edge_pass


_BLK = 1000
_GRID = (N // _BLK,)


@functools.lru_cache(maxsize=None)
def _make_mm_zr(dx):
    def body(x_ref, h_ref, wx_ref, wh_ref, h0_ref, tab_ref):
        r = (jnp.dot(x_ref[...], wx_ref[...],
                     preferred_element_type=jnp.float32)
             + jnp.dot(h_ref[...], wh_ref[...],
                       preferred_element_type=jnp.float32))
        h0_ref[0] = r[:, 0:64]
        h0_ref[1] = r[:, 64:128]
        tab_ref[0] = r[:, 128:256]
        tab_ref[1] = r[:, 256:384]

    return pl.pallas_call(
        body,
        grid=_GRID,
        in_specs=[
            pl.BlockSpec((_BLK, dx), lambda i: (i, 0)),
            pl.BlockSpec((_BLK, HID), lambda i: (i, 0)),
            pl.BlockSpec((dx, 384), lambda i: (0, 0)),
            pl.BlockSpec((HID, 384), lambda i: (0, 0)),
        ],
        out_specs=[
            pl.BlockSpec((2, _BLK, 64), lambda i: (0, i, 0)),
            pl.BlockSpec((2, _BLK, 128), lambda i: (0, i, 0)),
        ],
        out_shape=[
            jax.ShapeDtypeStruct((2, N, 64), jnp.float32),
            jax.ShapeDtypeStruct((2, N, 128), jnp.float32),
        ],
    )


@functools.lru_cache(maxsize=None)
def _make_mm_h(dx):
    def body(x_ref, h_ref, h0zr_ref, mzr_ref, bz_ref, br_ref, wx_ref,
             wh_ref, z_ref, h0h_ref, tab_ref):
        z = jax.nn.sigmoid(h0zr_ref[0] + mzr_ref[0] + bz_ref[...])
        r = jax.nn.sigmoid(h0zr_ref[1] + mzr_ref[1] + br_ref[...])
        g = (jnp.dot(x_ref[...], wx_ref[...],
                     preferred_element_type=jnp.float32)
             + jnp.dot(h_ref[...] * r, wh_ref[...],
                       preferred_element_type=jnp.float32))
        z_ref[...] = z
        h0h_ref[...] = g[:, 0:64]
        tab_ref[...] = g[:, 64:192]

    return pl.pallas_call(
        body,
        grid=_GRID,
        in_specs=[
            pl.BlockSpec((_BLK, dx), lambda i: (i, 0)),
            pl.BlockSpec((_BLK, HID), lambda i: (i, 0)),
            pl.BlockSpec((2, _BLK, 64), lambda i: (0, i, 0)),
            pl.BlockSpec((2, _BLK, 64), lambda i: (0, i, 0)),
            pl.BlockSpec((1, 64), lambda i: (0, 0)),
            pl.BlockSpec((1, 64), lambda i: (0, 0)),
            pl.BlockSpec((dx, 192), lambda i: (0, 0)),
            pl.BlockSpec((HID, 192), lambda i: (0, 0)),
        ],
        out_specs=[
            pl.BlockSpec((_BLK, 64), lambda i: (i, 0)),
            pl.BlockSpec((_BLK, 64), lambda i: (i, 0)),
            pl.BlockSpec((_BLK, 128), lambda i: (i, 0)),
        ],
        out_shape=[
            jax.ShapeDtypeStruct((N, 64), jnp.float32),
            jax.ShapeDtypeStruct((N, 64), jnp.float32),
            jax.ShapeDtypeStruct((N, 128), jnp.float32),
        ],
    )


def _mm_c_body(h0h_ref, mh_ref, bh_ref, z_ref, hprev_ref, out_ref):
    ht = jnp.tanh(h0h_ref[...] + mh_ref[...] + bh_ref[...])
    z = z_ref[...]
    out_ref[...] = jax.nn.relu(z * hprev_ref[...] + (1.0 - z) * ht)


@functools.lru_cache(maxsize=None)
def _make_mm_c():
    return pl.pallas_call(
        _mm_c_body,
        grid=_GRID,
        in_specs=[
            pl.BlockSpec((_BLK, 64), lambda i: (i, 0)),
            pl.BlockSpec((_BLK, 64), lambda i: (i, 0)),
            pl.BlockSpec((1, 64), lambda i: (0, 0)),
            pl.BlockSpec((_BLK, 64), lambda i: (i, 0)),
            pl.BlockSpec((_BLK, 64), lambda i: (i, 0)),
        ],
        out_specs=pl.BlockSpec((_BLK, 64), lambda i: (i, 0)),
        out_shape=jax.ShapeDtypeStruct((N, 64), jnp.float32),
    )


def _mm_lin_body(h_ref, w_ref, b_ref, out_ref):
    out_ref[...] = (jnp.dot(h_ref[...], w_ref[...],
                            preferred_element_type=jnp.float32)
                    + b_ref[...])


@functools.lru_cache(maxsize=None)
def _make_mm_lin():
    return pl.pallas_call(
        _mm_lin_body,
        grid=_GRID,
        in_specs=[
            pl.BlockSpec((_BLK, 64), lambda i: (i, 0)),
            pl.BlockSpec((64, 128), lambda i: (0, 0)),
            pl.BlockSpec((1, 128), lambda i: (0, 0)),
        ],
        out_specs=pl.BlockSpec((_BLK, 128), lambda i: (i, 0)),
        out_shape=jax.ShapeDtypeStruct((N, 128), jnp.float32),
    )


def _zr_weights(w_z, w_r):
    az = w_z[0, 0] + w_z[1, 0]
    ar = w_r[0, 0] + w_r[1, 0]
    cat = jnp.concatenate(
        [az, ar, w_z[0, 1], w_z[1, 1], w_r[0, 1], w_r[1, 1]], axis=1)
    return cat[:-HID], cat[-HID:]


def _h_weights(w_h):
    ah = w_h[0, 0] + w_h[1, 0]
    cat = jnp.concatenate([ah, w_h[0, 1], w_h[1, 1]], axis=1)
    return cat[:-HID], cat[-HID:]


def _cell(dx, x_in, h_prev, wzr_x, wzr_h, wh_x, wh_h, bz, br, bh,
          rowp, row2, colp, nop, nip):
    ep_zr = _make_edge_pass(128, False)
    ep_h = _make_edge_pass(128, True)
    h0zr, tabzr = _make_mm_zr(dx)(x_in, h_prev, wzr_x, wzr_h)
    mzr = ep_zr(tabzr.reshape(2 * N, 128), row2, colp, nop, nip)
    mzr = mzr.reshape(2, N, 64)
    z, h0h, tabh = _make_mm_h(dx)(x_in, h_prev, h0zr, mzr, bz, br,
                                  wh_x, wh_h)
    mh2 = ep_h(tabh, rowp, colp, nop, nip)
    mh = mh2[:N] + mh2[N:]
    return _make_mm_c()(h0h, mh, bh, z, h_prev)


def kernel(x, edge_index, edge_weight, w1_z, b1_z, w1_r, b1_r, w1_h, b1_h,
           w2_z, b2_z, w2_r, b2_r, w2_h, b2_h, lin_w, lin_b):
    row = edge_index[0]
    col = edge_index[1]
    deg_out = jnp.zeros((N,), jnp.float32).at[row].add(edge_weight)
    deg_in = jnp.zeros((N,), jnp.float32).at[col].add(edge_weight)
    no = (1.0 / deg_out)[row]
    ni = (1.0 / deg_in)[col]

    pad = E_PAD - E
    pad_idx = (jnp.arange(pad, dtype=jnp.int32) * 131) % N
    rowp = jnp.concatenate([row, pad_idx])
    colp = jnp.concatenate([col, pad_idx])
    nop = jnp.concatenate([no, jnp.zeros((pad,), jnp.float32)])
    nip = jnp.concatenate([ni, jnp.zeros((pad,), jnp.float32)])
    row2 = jnp.concatenate([rowp, rowp + N])

    wzr1_x, wzr1_h = _zr_weights(w1_z, w1_r)
    wh1_x, wh1_h = _h_weights(w1_h)
    wzr2_x, wzr2_h = _zr_weights(w2_z, w2_r)
    wh2_x, wh2_h = _h_weights(w2_h)
    bz1, br1, bh1 = (b.reshape(1, HID) for b in (b1_z, b1_r, b1_h))
    bz2, br2, bh2 = (b.reshape(1, HID) for b in (b2_z, b2_r, b2_h))

    xt_all = jnp.moveaxis(x[0], -1, 0)

    h1 = jnp.zeros((N, HID), jnp.float32)
    h2 = jnp.zeros((N, HID), jnp.float32)
    for t in range(T_STEPS):
        h1 = _cell(F_IN, xt_all[t], h1, wzr1_x, wzr1_h, wh1_x, wh1_h,
                   bz1, br1, bh1, rowp, row2, colp, nop, nip)
        h2 = _cell(HID, h1, h2, wzr2_x, wzr2_h, wh2_x, wh2_h,
                   bz2, br2, bh2, rowp, row2, colp, nop, nip)

    lwp = jnp.zeros((64, 128), jnp.float32).at[:, :12].set(lin_w.T)
    lbp = jnp.zeros((1, 128), jnp.float32).at[0, :12].set(lin_b)
    out = _make_mm_lin()(h2, lwp, lbp)
    return out[:, :12].reshape(1, N, 12)
